# baseline (speedup 1.0000x reference)
import functools

import jax
import jax.numpy as jnp
from jax import lax
from jax.experimental import pallas as pl
from jax.experimental.pallas import tpu as pltpu
from jax.experimental.pallas import tpu_sc as plsc

NUM_ROUTES = 8
HIDDEN = 768
FF = 1024
VOCAB = 1024
TB = 256
T = 2 * 2048
T_PAD = T + NUM_ROUTES * TB
NT = T_PAD // TB

NC, NS, L = 2, 16, 16
NW = NC * NS
CHUNK = T // NW
NV = CHUNK // L
NVTOT = T // L

_SC_MESH = plsc.VectorSubcoreMesh(core_axis_name="c", subcore_axis_name="s")


@functools.partial(
    pl.kernel,
    mesh=_SC_MESH,
    out_type=[
        jax.ShapeDtypeStruct((T_PAD, HIDDEN), jnp.float32),
        jax.ShapeDtypeStruct((T,), jnp.int32),
        jax.ShapeDtypeStruct((32,), jnp.int32),
        jax.ShapeDtypeStruct((16,), jnp.int32),
    ],
    scratch_types=[
        pltpu.VMEM((T,), jnp.int32),
        pltpu.VMEM((CHUNK, HIDDEN), jnp.float32),
        pltpu.VMEM((CHUNK,), jnp.int32),
        pltpu.VMEM((16,), jnp.int32),
        pltpu.VMEM((16,), jnp.int32),
        pltpu.VMEM((32,), jnp.int32),
        pltpu.VMEM((16,), jnp.int32),
        pltpu.SemaphoreType.DMA,
    ],
    compiler_params=pltpu.CompilerParams(needs_layout_passes=False),
)
def _sc_route_scatter(rid_hbm, x_hbm, xs_hbm, inv_hbm, rt_hbm, nact_hbm,
                      ridv, xbuf, invv, basebuf, endbuf, rtbuf, nactbuf, xsem):
    wid = lax.axis_index("s") * NC + lax.axis_index("c")
    base = wid * CHUNK
    xcp = pltpu.async_copy(x_hbm.at[pl.ds(base, CHUNK)], xbuf, xsem)
    pltpu.sync_copy(rid_hbm, ridv)
    lanes = lax.iota(jnp.int32, 16)
    myvec0 = wid * NV

    def count_body(i, carry):
        tot_v, pre_v = carry
        vec = ridv[pl.ds(i * L, L)]
        ipre = (i < myvec0).astype(jnp.int32)
        for e in range(NUM_ROUTES):
            cnt = plsc.all_reduce_population_count(vec == e)
            sel = jnp.where(lanes == e, cnt, 0)
            tot_v = tot_v + sel
            pre_v = pre_v + sel * ipre
        return tot_v, pre_v

    z = jnp.zeros((16,), jnp.int32)
    tot_v, pre_v = lax.fori_loop(0, NVTOT, count_body, (z, z))
    g_pad_v = ((tot_v + TB - 1) // TB) * TB
    cums = plsc.cumsum(g_pad_v)
    start_v = cums - g_pad_v
    basebuf[...] = start_v + pre_v
    endbuf[...] = cums // TB

    @pl.when(wid == 0)
    def _():
        endv = endbuf[...]
        ends = [endv[e] for e in range(NUM_ROUTES)]
        for k in range(2):
            tvec = lax.iota(jnp.int32, 16) + k * 16
            r = jnp.zeros((16,), jnp.int32)
            for e in range(NUM_ROUTES):
                r = r + (tvec >= ends[e]).astype(jnp.int32)
            rtbuf[pl.ds(k * 16, 16)] = jnp.minimum(r, NUM_ROUTES - 1)
        nactbuf[...] = jnp.where(lanes == 0, ends[NUM_ROUTES - 1], 0)
        pltpu.sync_copy(rtbuf, rt_hbm)
        pltpu.sync_copy(nactbuf, nact_hbm)

    basev = basebuf[...]
    runs = [basev[e] for e in range(NUM_ROUTES)]
    for v in range(NV):
        vec = ridv[pl.ds((myvec0 + v) * L, L)]
        inv_vec = jnp.zeros((16,), jnp.int32)
        for e in range(NUM_ROUTES):
            m = vec == e
            csum = plsc.cumsum(jnp.where(m, jnp.int32(1), jnp.int32(0)))
            inv_vec = jnp.where(m, runs[e] + csum - 1, inv_vec)
            runs[e] = runs[e] + jnp.max(csum)
        invv[pl.ds(v * L, L)] = inv_vec
    pltpu.sync_copy(invv, inv_hbm.at[pl.ds(base, CHUNK)])
    xcp.wait()
    pltpu.async_copy(xbuf, xs_hbm.at[invv], xsem).wait()


@functools.partial(
    pl.kernel,
    mesh=_SC_MESH,
    out_type=jax.ShapeDtypeStruct((T,), jnp.float32),
    scratch_types=[
        pltpu.VMEM((NT, 1, TB), jnp.float32),
        pltpu.VMEM((CHUNK,), jnp.int32),
        pltpu.VMEM((CHUNK,), jnp.float32),
    ],
    compiler_params=pltpu.CompilerParams(needs_layout_passes=False),
)
def _sc_gather_probs(inv_hbm, out3_hbm, probs_hbm, obuf, invv, pbuf):
    wid = lax.axis_index("s") * NC + lax.axis_index("c")
    base = wid * CHUNK
    pltpu.sync_copy(inv_hbm.at[pl.ds(base, CHUNK)], invv)
    pltpu.sync_copy(out3_hbm, obuf)
    zero = jnp.zeros((16,), jnp.int32)
    for v in range(NV):
        idx = invv[pl.ds(v * L, L)]
        pbuf[pl.ds(v * L, L)] = plsc.load_gather(
            obuf, [idx // TB, zero, idx % TB])
    pltpu.sync_copy(pbuf, probs_hbm.at[pl.ds(base, CHUNK)])


def _tc_body(route_ref, nact_ref, x_ref, w1a_ref, w1b_ref, b1_ref,
             w2a_ref, w2b_ref, w2c_ref, w2d_ref, b2_ref, out_ref):
    i = pl.program_id(0)

    @pl.when(i < nact_ref[0])
    def _():
        x = x_ref[...]
        h = jnp.concatenate(
            [jnp.dot(x, w1a_ref[0], preferred_element_type=jnp.float32),
             jnp.dot(x, w1b_ref[0], preferred_element_type=jnp.float32)],
            axis=1)
        h = jax.nn.gelu(h + b1_ref[0, 0])
        VQ = VOCAB // 4
        ls = [jnp.dot(h, w_ref[0], preferred_element_type=jnp.float32)
              + b2_ref[0, 0, pl.ds(j * VQ, VQ)]
              for j, w_ref in enumerate((w2a_ref, w2b_ref, w2c_ref, w2d_ref))]
        m = jnp.maximum(jnp.maximum(jnp.max(ls[0], axis=-1), jnp.max(ls[1], axis=-1)),
                        jnp.maximum(jnp.max(ls[2], axis=-1), jnp.max(ls[3], axis=-1)))
        s = sum(jnp.sum(jnp.exp(l - m[:, None]), axis=-1) for l in ls)
        out_ref[0, 0, :] = 1.0 / s


def _grouped_heads(x_sorted, route_of_tile, nactive, W1, b1, W2, b2):
    FH = FF // 2
    VQ = VOCAB // 4
    grid_spec = pltpu.PrefetchScalarGridSpec(
        num_scalar_prefetch=2,
        grid=(NT,),
        in_specs=[
            pl.BlockSpec((TB, HIDDEN),
                         lambda i, rt, na: (jnp.minimum(i, na[0] - 1), 0)),
            pl.BlockSpec((1, HIDDEN, FH), lambda i, rt, na: (rt[i], 0, 0)),
            pl.BlockSpec((1, HIDDEN, FH), lambda i, rt, na: (rt[i], 0, 1)),
            pl.BlockSpec((1, 1, FF), lambda i, rt, na: (rt[i], 0, 0)),
            pl.BlockSpec((1, FF, VQ), lambda i, rt, na: (rt[i], 0, 0)),
            pl.BlockSpec((1, FF, VQ), lambda i, rt, na: (rt[i], 0, 1)),
            pl.BlockSpec((1, FF, VQ), lambda i, rt, na: (rt[i], 0, 2)),
            pl.BlockSpec((1, FF, VQ), lambda i, rt, na: (rt[i], 0, 3)),
            pl.BlockSpec((1, 1, VOCAB), lambda i, rt, na: (rt[i], 0, 0)),
        ],
        out_specs=pl.BlockSpec((1, 1, TB), lambda i, rt, na: (i, 0, 0)),
    )
    out3 = pl.pallas_call(
        _tc_body,
        grid_spec=grid_spec,
        out_shape=jax.ShapeDtypeStruct((NT, 1, TB), jnp.float32),
    )(route_of_tile, nactive, x_sorted,
      W1, W1, b1.reshape(NUM_ROUTES, 1, FF),
      W2, W2, W2, W2, b2.reshape(NUM_ROUTES, 1, VOCAB))
    return out3


def kernel(e_two, batch_route_ids, W1, b1, W2, b2):
    Bb, Ss, H = e_two.shape
    x = e_two.reshape(T, H)
    rid = batch_route_ids.reshape(T).astype(jnp.int32)

    x_sorted, inv_pos, route_of_tile, nactive = _sc_route_scatter(rid, x)

    out3 = _grouped_heads(x_sorted, route_of_tile, nactive, W1, b1, W2, b2)

    probs = _sc_gather_probs(inv_pos, out3)
    return probs.reshape(Bb, Ss)


# speedup vs baseline: 1.0180x; 1.0180x over previous
import functools

import jax
import jax.numpy as jnp
from jax import lax
from jax.experimental import pallas as pl
from jax.experimental.pallas import tpu as pltpu
from jax.experimental.pallas import tpu_sc as plsc

NUM_ROUTES = 8
HIDDEN = 768
FF = 1024
VOCAB = 1024
TB = 256
T = 2 * 2048
T_PAD = T + NUM_ROUTES * TB
NT = T_PAD // TB

NC, NS, L = 2, 16, 16
NW = NC * NS
CHUNK = T // NW
NV = CHUNK // L
NVTOT = T // L

_SC_MESH = plsc.VectorSubcoreMesh(core_axis_name="c", subcore_axis_name="s")


@functools.partial(
    pl.kernel,
    mesh=_SC_MESH,
    out_type=[
        jax.ShapeDtypeStruct((T_PAD, HIDDEN), jnp.float32),
        jax.ShapeDtypeStruct((T,), jnp.int32),
        jax.ShapeDtypeStruct((32,), jnp.int32),
        jax.ShapeDtypeStruct((16,), jnp.int32),
    ],
    scratch_types=[
        pltpu.VMEM((T,), jnp.int32),
        [pltpu.VMEM((CHUNK // 4, HIDDEN), jnp.float32) for _ in range(4)],
        pltpu.VMEM((CHUNK,), jnp.int32),
        [pltpu.VMEM((CHUNK // 4,), jnp.int32) for _ in range(4)],
        pltpu.VMEM((16,), jnp.int32),
        pltpu.VMEM((16,), jnp.int32),
        pltpu.VMEM((32,), jnp.int32),
        pltpu.VMEM((16,), jnp.int32),
        pltpu.SemaphoreType.DMA,
        pltpu.SemaphoreType.DMA,
    ],
    compiler_params=pltpu.CompilerParams(needs_layout_passes=False),
)
def _sc_route_scatter(rid_hbm, x_hbm, xs_hbm, inv_hbm, rt_hbm, nact_hbm,
                      ridv, xbufs, invv, invcs, basebuf, endbuf, rtbuf, nactbuf,
                      rsem, wsem):
    wid = lax.axis_index("s") * NC + lax.axis_index("c")
    base = wid * CHUNK
    QC = CHUNK // 4
    xcps = [pltpu.async_copy(x_hbm.at[pl.ds(base + q * QC, QC)], xbufs[q], rsem)
            for q in range(4)]
    pltpu.sync_copy(rid_hbm, ridv)
    lanes = lax.iota(jnp.int32, 16)
    myvec0 = wid * NV

    def count_body(i, carry):
        tot_v, pre_v = carry
        vec = ridv[pl.ds(i * L, L)]
        ipre = (i < myvec0).astype(jnp.int32)
        for e in range(NUM_ROUTES):
            cnt = plsc.all_reduce_population_count(vec == e)
            sel = jnp.where(lanes == e, cnt, 0)
            tot_v = tot_v + sel
            pre_v = pre_v + sel * ipre
        return tot_v, pre_v

    z = jnp.zeros((16,), jnp.int32)
    tot_v, pre_v = lax.fori_loop(0, NVTOT, count_body, (z, z))
    g_pad_v = ((tot_v + TB - 1) // TB) * TB
    cums = plsc.cumsum(g_pad_v)
    start_v = cums - g_pad_v
    basebuf[...] = start_v + pre_v
    endbuf[...] = cums // TB

    @pl.when(wid == 0)
    def _():
        endv = endbuf[...]
        ends = [endv[e] for e in range(NUM_ROUTES)]
        for k in range(2):
            tvec = lax.iota(jnp.int32, 16) + k * 16
            r = jnp.zeros((16,), jnp.int32)
            for e in range(NUM_ROUTES):
                r = r + (tvec >= ends[e]).astype(jnp.int32)
            rtbuf[pl.ds(k * 16, 16)] = jnp.minimum(r, NUM_ROUTES - 1)
        nactbuf[...] = jnp.where(lanes == 0, ends[NUM_ROUTES - 1], 0)
        pltpu.sync_copy(rtbuf, rt_hbm)
        pltpu.sync_copy(nactbuf, nact_hbm)

    basev = basebuf[...]
    runs = [basev[e] for e in range(NUM_ROUTES)]
    for v in range(NV):
        vec = ridv[pl.ds((myvec0 + v) * L, L)]
        inv_vec = jnp.zeros((16,), jnp.int32)
        for e in range(NUM_ROUTES):
            m = vec == e
            csum = plsc.cumsum(jnp.where(m, jnp.int32(1), jnp.int32(0)))
            inv_vec = jnp.where(m, runs[e] + csum - 1, inv_vec)
            runs[e] = runs[e] + jnp.max(csum)
        invv[pl.ds(v * L, L)] = inv_vec
        invcs[v // 2][pl.ds((v % 2) * L, L)] = inv_vec
    pltpu.sync_copy(invv, inv_hbm.at[pl.ds(base, CHUNK)])
    scps = []
    for q in range(4):
        xcps[q].wait()
        scps.append(pltpu.async_copy(xbufs[q], xs_hbm.at[invcs[q]], wsem))
    for cp in scps:
        cp.wait()


@functools.partial(
    pl.kernel,
    mesh=_SC_MESH,
    out_type=jax.ShapeDtypeStruct((T,), jnp.float32),
    scratch_types=[
        pltpu.VMEM((T_PAD,), jnp.float32),
        pltpu.VMEM((CHUNK,), jnp.int32),
        pltpu.VMEM((CHUNK,), jnp.float32),
    ],
    compiler_params=pltpu.CompilerParams(needs_layout_passes=False),
)
def _sc_gather_probs(inv_hbm, outflat_hbm, probs_hbm, obuf, invv, pbuf):
    wid = lax.axis_index("s") * NC + lax.axis_index("c")
    base = wid * CHUNK
    pltpu.sync_copy(inv_hbm.at[pl.ds(base, CHUNK)], invv)
    pltpu.sync_copy(outflat_hbm, obuf)
    for v in range(NV):
        idx = invv[pl.ds(v * L, L)]
        pbuf[pl.ds(v * L, L)] = plsc.load_gather(obuf, [idx])
    pltpu.sync_copy(pbuf, probs_hbm.at[pl.ds(base, CHUNK)])


def _tc_body(route_ref, nact_ref, x_ref, w1_ref, b1_ref, w2_ref, b2_ref, out_ref):
    i = pl.program_id(0)

    @pl.when(i < nact_ref[0])
    def _():
        h = jnp.dot(x_ref[...], w1_ref[0], preferred_element_type=jnp.float32)
        h = jax.nn.gelu(h + b1_ref[0, 0])
        logits = jnp.dot(h, w2_ref[0], preferred_element_type=jnp.float32)
        logits = logits + b2_ref[0, 0]
        m = jnp.max(logits, axis=-1)
        s = jnp.sum(jnp.exp(logits - m[:, None]), axis=-1)
        out_ref[...] = 1.0 / s


def _grouped_heads(x_sorted, route_of_tile, nactive, W1, b1, W2, b2):
    grid_spec = pltpu.PrefetchScalarGridSpec(
        num_scalar_prefetch=2,
        grid=(NT,),
        in_specs=[
            pl.BlockSpec((TB, HIDDEN),
                         lambda i, rt, na: (jnp.minimum(i, na[0] - 1), 0)),
            pl.BlockSpec((1, HIDDEN, FF), lambda i, rt, na: (rt[i], 0, 0)),
            pl.BlockSpec((1, 1, FF), lambda i, rt, na: (rt[i], 0, 0)),
            pl.BlockSpec((1, FF, VOCAB), lambda i, rt, na: (rt[i], 0, 0)),
            pl.BlockSpec((1, 1, VOCAB), lambda i, rt, na: (rt[i], 0, 0)),
        ],
        out_specs=pl.BlockSpec((TB,), lambda i, rt, na: (i,)),
    )
    out_flat = pl.pallas_call(
        _tc_body,
        grid_spec=grid_spec,
        out_shape=jax.ShapeDtypeStruct((T_PAD,), jnp.float32),
    )(route_of_tile, nactive, x_sorted,
      W1, b1.reshape(NUM_ROUTES, 1, FF), W2, b2.reshape(NUM_ROUTES, 1, VOCAB))
    return out_flat


def kernel(e_two, batch_route_ids, W1, b1, W2, b2):
    Bb, Ss, H = e_two.shape
    x = e_two.reshape(T, H)
    rid = batch_route_ids.reshape(T).astype(jnp.int32)

    x_sorted, inv_pos, route_of_tile, nactive = _sc_route_scatter(rid, x)

    out3 = _grouped_heads(x_sorted, route_of_tile, nactive, W1, b1, W2, b2)

    probs = _sc_gather_probs(inv_pos, out3)
    return probs.reshape(Bb, Ss)
